# baseline (speedup 1.0000x reference)
import functools
import math

import jax
import jax.numpy as jnp
from jax import lax
from jax.experimental import pallas as pl
from jax.experimental.pallas import tpu as pltpu
from jax.experimental.pallas import tpu_sc as plsc

NC = 2
NS = 16
NW = NC * NS
CHUNK = 128
NBUF = 4


def _gather_kernel(B, V, D):
  b_per_w = B // NW
  n_ch = b_per_w // CHUNK
  mesh = plsc.VectorSubcoreMesh(core_axis_name="c", subcore_axis_name="s")

  @functools.partial(
      pl.kernel,
      mesh=mesh,
      compiler_params=pltpu.CompilerParams(use_tc_tiling_on_sc=False),
      out_type=jax.ShapeDtypeStruct((B, D), jnp.float32),
      scratch_types=[
          pltpu.VMEM((n_ch, CHUNK), jnp.int32),
          pltpu.VMEM((NBUF, CHUNK, D), jnp.float32),
          [pltpu.SemaphoreType.DMA] * NBUF,
          [pltpu.SemaphoreType.DMA] * NBUF,
      ],
  )
  def k(idx_hbm, table_hbm, out_hbm, idx_v, rbuf, gsem, ssem):
    wid = lax.axis_index("s") * NC + lax.axis_index("c")
    base = wid * b_per_w
    pltpu.sync_copy(idx_hbm.at[wid], idx_v)

    for b in range(NBUF):
      pltpu.async_copy(table_hbm.at[idx_v.at[b]], rbuf.at[b], gsem[b])

    @pl.loop(0, n_ch, step=NBUF)
    def _grp(c0):
      for b in range(NBUF):
        c = c0 + b
        pltpu.make_async_copy(
            table_hbm.at[idx_v.at[b]], rbuf.at[b], gsem[b]).wait()
        pltpu.async_copy(
            rbuf.at[b], out_hbm.at[pl.ds(base + c * CHUNK, CHUNK)], ssem[b])

        @pl.when(c + NBUF < n_ch)
        def _():
          pltpu.make_async_copy(
              rbuf.at[b], out_hbm.at[pl.ds(base, CHUNK)], ssem[b]).wait()
          pltpu.async_copy(
              table_hbm.at[idx_v.at[c + NBUF]], rbuf.at[b], gsem[b])

    for b in range(NBUF):
      pltpu.make_async_copy(
          rbuf.at[b], out_hbm.at[pl.ds(base, CHUNK)], ssem[b]).wait()

  return k


def kernel(tokens, table):
  B0, S = tokens.shape
  V, D = table.shape
  B = B0 * S
  idx = jnp.clip(tokens.astype(jnp.int32), 0, V - 1)
  idx = idx.reshape(NW, (B // NW) // CHUNK, CHUNK)
  raw = _gather_kernel(B, V, D)(idx, table)
  return raw.reshape(B0, S, D) * math.sqrt(D)
